# baseline (speedup 1.0000x reference)
import functools
import jax
import jax.numpy as jnp
from jax import lax
from jax.experimental import pallas as pl
from jax.experimental.pallas import tpu as pltpu
from jax.experimental.pallas import tpu_sc as plsc


def kernel(x, shape, row_embed, col_embed):
    b, _, h, w = x.shape
    f = row_embed.shape[1]
    hw = h * w
    nlane = 16
    nchunk = f // nlane

    mesh = plsc.VectorSubcoreMesh(core_axis_name="c", subcore_axis_name="s")
    nc = mesh.num_cores

    @functools.partial(
        pl.kernel,
        out_type=jax.ShapeDtypeStruct((b, hw, 2 * f), jnp.float32),
        mesh=mesh,
        scratch_types=[
            pltpu.VMEM((nlane,), jnp.int32),
            pltpu.VMEM((w, 2 * f), jnp.float32),
            pltpu.VMEM((f,), jnp.float32),
            pltpu.SemaphoreType.DMA,
        ],
    )
    def sc_kernel(shape_hbm, row_hbm, col_hbm, out_hbm, shape_v, tile, row1_v, sem):
        wid = lax.axis_index("s") * nc + lax.axis_index("c")

        pltpu.sync_copy(shape_hbm, shape_v)
        pltpu.sync_copy(row_hbm.at[wid], row1_v)
        pltpu.sync_copy(col_hbm.at[pl.ds(0, w)], tile.at[:, pl.ds(0, f)])

        sv = shape_v[...]
        residual = (sv[2] * sv[3] - hw).astype(jnp.float32)
        resv = jnp.full((nlane,), residual, jnp.float32)

        rcs = [row1_v[pl.ds(j * nlane, nlane)] + resv for j in range(nchunk)]
        for i in range(w):
            for j in range(nchunk):
                tile[i, pl.ds(f + j * nlane, nlane)] = rcs[j]
        for i in range(w):
            for j in range(nchunk):
                sl = pl.ds(j * nlane, nlane)
                tile[i, sl] = tile[i, sl] + resv

        row0 = wid * w
        copies = [
            pltpu.async_copy(tile, out_hbm.at[bi, pl.ds(row0, w), :], sem)
            for bi in range(b)
        ]
        for cp in copies:
            cp.wait()

    shape16 = jnp.tile(shape.astype(jnp.int32), 4)
    return sc_kernel(shape16, row_embed, col_embed)


# speedup vs baseline: 4.6446x; 4.6446x over previous
import jax
import jax.numpy as jnp
from jax.experimental import pallas as pl
from jax.experimental.pallas import tpu as pltpu


def kernel(x, shape, row_embed, col_embed):
    b, _, h, w = x.shape
    f = row_embed.shape[1]
    hw = h * w
    nsplit = 2
    rows = hw // nsplit

    def body(shape_ref, col_ref, row_ref, out_ref, pos_ref, sem):
        residual = (shape_ref[2] * shape_ref[3] - hw).astype(jnp.float32)
        col = col_ref[...]
        row = row_ref[...]
        pos_ref[:, :f] = jnp.broadcast_to(col[None], (h, w, f)).reshape(hw, f) + residual
        pos_ref[:, f:] = jnp.broadcast_to(row[:, None], (h, w, f)).reshape(hw, f) + residual
        copies = [
            pltpu.make_async_copy(
                pos_ref.at[pl.ds(s * rows, rows)],
                out_ref.at[i, pl.ds(s * rows, rows), :],
                sem.at[i * nsplit + s],
            )
            for i in range(b)
            for s in range(nsplit)
        ]
        for c in copies:
            c.start()
        for c in copies:
            c.wait()

    grid_spec = pltpu.PrefetchScalarGridSpec(
        num_scalar_prefetch=1,
        grid=(1,),
        in_specs=[
            pl.BlockSpec((w, f), lambda i, s: (0, 0)),
            pl.BlockSpec((h, f), lambda i, s: (0, 0)),
        ],
        out_specs=pl.BlockSpec(memory_space=pl.ANY),
        scratch_shapes=[
            pltpu.VMEM((hw, 2 * f), jnp.float32),
            pltpu.SemaphoreType.DMA((b * nsplit,)),
        ],
    )

    return pl.pallas_call(
        body,
        grid_spec=grid_spec,
        out_shape=jax.ShapeDtypeStruct((b, hw, 2 * f), jnp.float32),
    )(shape, col_embed, row_embed)
